# baseline (speedup 1.0000x reference)
import functools

import jax
import jax.numpy as jnp
from jax import lax
from jax.experimental import pallas as pl
from jax.experimental.pallas import tpu as pltpu
from jax.experimental.pallas import tpu_sc as plsc

NUM_CORES = 2
NUM_SUBCORES = 16
NUM_WORKERS = NUM_CORES * NUM_SUBCORES
BLK = 128
PDIM = 128
NBUF = 7
LEAD = 5


@functools.partial(jax.jit, static_argnames=("total", "dim", "seq", "bpl", "nblk"))
def _gather_sc(idx4, table_p, *, total, dim, seq, bpl, nblk):
    mesh = plsc.VectorSubcoreMesh(
        core_axis_name="c", subcore_axis_name="s",
        num_cores=NUM_CORES, num_subcores=NUM_SUBCORES)

    @functools.partial(
        pl.kernel,
        out_type=(jax.ShapeDtypeStruct((total, dim), table_p.dtype),
                  jax.ShapeDtypeStruct((BLK, dim), table_p.dtype)),
        mesh=mesh,
        compiler_params=pltpu.CompilerParams(use_tc_tiling_on_sc=False),
        scratch_types=[
            pltpu.VMEM((nblk, BLK), jnp.int32),
            pltpu.VMEM((NBUF, BLK, PDIM), table_p.dtype),
        ] + [pltpu.SemaphoreType.DMA] * (2 * NBUF),
    )
    def body(idx_hbm, table_hbm, out_hbm, dump_hbm, idx_v, rows_v, *sems):
        gsems = sems[:NBUF]
        wsems = sems[NBUF:]
        wid = lax.axis_index("s") * NUM_CORES + lax.axis_index("c")
        pltpu.sync_copy(idx_hbm.at[wid], idx_v)

        def gather(j, bb):
            return pltpu.make_async_copy(
                table_hbm.at[idx_v.at[j]], rows_v.at[bb], gsems[bb])

        def decode(j):
            p = wid * nblk + j
            tr = p // (bpl * 8)
            rm = p % (bpl * 8)
            tc = rm // 8
            s = rm % 8
            return tr * 8 + s, tc

        def src(bb):
            return rows_v.at[bb].at[:, pl.ds(0, dim)]

        def wb_fire(j, bb):
            l, tc = decode(j)

            @pl.when(l < seq)
            def _():
                pltpu.make_async_copy(
                    src(bb), out_hbm.at[pl.ds((l * bpl + tc) * BLK, BLK)],
                    wsems[bb]).start()

            @pl.when(l >= seq)
            def _():
                pltpu.make_async_copy(src(bb), dump_hbm, wsems[bb]).start()

        def wb_drain(bb):
            pltpu.make_async_copy(src(bb), dump_hbm, wsems[bb]).wait()

        for k in range(LEAD):
            gather(k, k).start()

        def step(go, carry):
            for bb in range(NBUF):
                j = go * NBUF + bb
                gather(j, bb).wait()
                wb_fire(j, bb)
                nxt = j + LEAD

                @pl.when(nxt < nblk)
                def _():
                    bn = (bb + LEAD) % NBUF

                    @pl.when(nxt - NBUF >= 0)
                    def _():
                        wb_drain(bn)

                    gather(nxt, bn).start()
            return carry

        lax.fori_loop(0, nblk // NBUF, step, 0)
        for k in range(NBUF):
            wb_drain((nblk - NBUF + k) % NBUF)

    return body(idx4, table_p)


def kernel(indices, table):
    bsz, seq = indices.shape
    dim = table.shape[1]
    total = bsz * seq
    seq_p = ((seq + 7) // 8) * 8
    ntr = seq_p // 8
    bpl = bsz // BLK
    assert bsz % BLK == 0 and dim <= PDIM
    nblocks = ntr * bpl * 8
    assert nblocks % (NUM_WORKERS * NBUF) == 0
    nblk = nblocks // NUM_WORKERS
    idx_p = jnp.pad(indices.astype(jnp.int32), ((0, 0), (0, seq_p - seq)))
    idx4 = (idx_p.T.reshape(ntr, 8, bpl, BLK)
            .transpose(0, 2, 1, 3)
            .reshape(NUM_WORKERS, nblk, BLK))
    table_p = jnp.pad(table, ((0, 0), (0, PDIM - dim)))
    out, _ = _gather_sc(idx4, table_p, total=total, dim=dim, seq=seq,
                        bpl=bpl, nblk=nblk)
    return out.reshape(seq, bsz, dim).transpose(1, 0, 2)


# speedup vs baseline: 2.1795x; 2.1795x over previous
import functools

import jax
import jax.numpy as jnp
from jax import lax
from jax.experimental import pallas as pl
from jax.experimental.pallas import tpu as pltpu
from jax.experimental.pallas import tpu_sc as plsc

NUM_CORES = 2
NUM_SUBCORES = 16
NUM_WORKERS = NUM_CORES * NUM_SUBCORES
CHUNK = 128
GS = 5
NBUF = 2


@functools.partial(jax.jit, static_argnames=("total", "dim", "nchunk"))
def _gather_sc(idx, table, *, total, dim, nchunk):
    ngrp = nchunk // GS
    grows = GS * CHUNK
    mesh = plsc.VectorSubcoreMesh(
        core_axis_name="c", subcore_axis_name="s",
        num_cores=NUM_CORES, num_subcores=NUM_SUBCORES)

    @functools.partial(
        pl.kernel,
        out_type=jax.ShapeDtypeStruct((total, dim), table.dtype),
        mesh=mesh,
        compiler_params=pltpu.CompilerParams(use_tc_tiling_on_sc=False),
        scratch_types=[
            pltpu.VMEM((nchunk, CHUNK), jnp.int32),
            pltpu.VMEM((NBUF, grows, dim), table.dtype),
            pltpu.SemaphoreType.DMA,
            pltpu.SemaphoreType.DMA,
            pltpu.SemaphoreType.DMA,
            pltpu.SemaphoreType.DMA,
        ],
    )
    def body(idx_hbm, table_hbm, out_hbm, idx_v, rows_v, g0, g1, w0, w1):
        gsems = (g0, g1)
        wsems = (w0, w1)
        wid = lax.axis_index("s") * NUM_CORES + lax.axis_index("c")
        base = wid * (nchunk * CHUNK)
        pltpu.sync_copy(idx_hbm.at[wid], idx_v)

        def fire(g, gb):
            for c in range(GS):
                pltpu.async_copy(
                    table_hbm.at[idx_v.at[g * GS + c]],
                    rows_v.at[gb].at[pl.ds(c * CHUNK, CHUNK)],
                    gsems[gb])

        def drain(g, gb):
            for c in range(GS):
                pltpu.make_async_copy(
                    table_hbm.at[idx_v.at[g * GS + c]],
                    rows_v.at[gb].at[pl.ds(c * CHUNK, CHUNK)],
                    gsems[gb]).wait()

        def wb(g, gb):
            return pltpu.make_async_copy(
                rows_v.at[gb], out_hbm.at[pl.ds(base + g * grows, grows)],
                wsems[gb])

        fire(0, 0)

        def step(go, carry):
            for gg in range(NBUF):
                g = go * NBUF + gg
                nxt = g + 1
                @pl.when(nxt < ngrp)
                def _():
                    @pl.when(g >= 1)
                    def _():
                        wb(g - 1, 1 - gg).wait()
                    fire(nxt, 1 - gg)

                drain(g, gg)
                wb(g, gg).start()
            return carry

        lax.fori_loop(0, ngrp // NBUF, step, 0)
        for gg in range(NBUF):
            wb(ngrp - NBUF + gg, gg).wait()

    return body(idx, table)


def kernel(indices, table):
    bsz, seq = indices.shape
    dim = table.shape[1]
    total = bsz * seq
    assert total % (NUM_WORKERS * CHUNK * GS * NBUF) == 0
    nchunk = total // (NUM_WORKERS * CHUNK)
    idx = indices.astype(jnp.int32).T.reshape(NUM_WORKERS, nchunk, CHUNK)
    out = _gather_sc(idx, table, total=total, dim=dim, nchunk=nchunk)
    return out.reshape(seq, bsz, dim).transpose(1, 0, 2)
